# Initial kernel scaffold; baseline (speedup 1.0000x reference)
import jax
import jax.numpy as jnp
from jax.experimental import pallas as pl


def kernel(coordinates, grid):
    raise NotImplementedError("write your pallas kernel here")


# speedup vs baseline: 1.4133x; 1.4133x over previous
import functools

import jax
import jax.numpy as jnp
from jax import lax
from jax.experimental import pallas as pl
from jax.experimental.pallas import tpu as pltpu
from jax.experimental.pallas import tpu_sc as plsc

C = 32
D = H = W = 96
DHW = D * H * W
N = 1048576
NW = 32
PPW = N // NW
B = 128
NCHUNK = PPW // B
LANES = 16
NGRP = B // LANES


def _interp_kernel(cz, cy, cx, table):
    mesh = plsc.VectorSubcoreMesh(core_axis_name="c", subcore_axis_name="s")

    @functools.partial(
        pl.kernel,
        out_type=jax.ShapeDtypeStruct((N, C), jnp.float32),
        mesh=mesh,
        compiler_params=pltpu.CompilerParams(
            needs_layout_passes=False, use_tc_tiling_on_sc=False
        ),
        scratch_types=(
            [pltpu.VMEM((B,), jnp.float32) for _ in range(3)]
            + [pltpu.VMEM((B,), jnp.float32) for _ in range(3)]
            + [pltpu.VMEM((B,), jnp.int32) for _ in range(8)]
            + [pltpu.VMEM((B, C), jnp.float32) for _ in range(8)]
            + [
                pltpu.VMEM((B, C), jnp.float32),
                pltpu.SemaphoreType.DMA,
            ]
        ),
    )
    def body(cz_hbm, cy_hbm, cx_hbm, table_hbm, out_hbm, *scratch):
        czv, cyv, cxv = scratch[0:3]
        wzv, wyv, wxv = scratch[3:6]
        idx = scratch[6:14]
        corners = scratch[14:22]
        obuf = scratch[22]
        sem = scratch[23]

        wid = lax.axis_index("s") * 2 + lax.axis_index("c")
        lanes = lax.iota(jnp.int32, LANES)

        def chunk_body(ch, _):
            pbase = wid * PPW + ch * B
            pltpu.sync_copy(cz_hbm.at[pl.ds(pbase, B)], czv)
            pltpu.sync_copy(cy_hbm.at[pl.ds(pbase, B)], cyv)
            pltpu.sync_copy(cx_hbm.at[pl.ds(pbase, B)], cxv)

            def build(g, _):
                s = g * LANES
                zs = jnp.clip(czv[pl.ds(s, LANES)] * D - 0.5, 0.0, D - 1.0)
                ys = jnp.clip(cyv[pl.ds(s, LANES)] * H - 0.5, 0.0, H - 1.0)
                xs = jnp.clip(cxv[pl.ds(s, LANES)] * W - 0.5, 0.0, W - 1.0)
                z0 = zs.astype(jnp.int32)
                y0 = ys.astype(jnp.int32)
                x0 = xs.astype(jnp.int32)
                wzv[pl.ds(s, LANES)] = zs - z0.astype(jnp.float32)
                wyv[pl.ds(s, LANES)] = ys - y0.astype(jnp.float32)
                wxv[pl.ds(s, LANES)] = xs - x0.astype(jnp.float32)
                dz = jnp.where(z0 < D - 1, H * W, 0)
                dy = jnp.where(y0 < H - 1, W, 0)
                dx = jnp.where(x0 < W - 1, 1, 0)
                base = z0 * (H * W) + y0 * W + x0
                for c in range(8):
                    off = base
                    if c & 4:
                        off = off + dz
                    if c & 2:
                        off = off + dy
                    if c & 1:
                        off = off + dx
                    idx[c][pl.ds(s, LANES)] = off
                return 0

            lax.fori_loop(0, NGRP, build, 0, unroll=2)

            copies = [
                pltpu.async_copy(table_hbm.at[idx[c]], corners[c], sem)
                for c in range(8)
            ]
            for cp in copies:
                cp.wait()

            def combine(g, _):
                s = g * LANES
                pt = s + lanes
                wz = wzv[pl.ds(s, LANES)]
                wy = wyv[pl.ds(s, LANES)]
                wx = wxv[pl.ds(s, LANES)]
                uz = 1.0 - wz
                uy = 1.0 - wy
                ux = 1.0 - wx
                wzy = [uz * uy, uz * wy, wz * uy, wz * wy]
                ws = []
                for c in range(8):
                    ws.append(wzy[c >> 1] * (wx if c & 1 else ux))
                for k in range(C):
                    kv = jnp.full((LANES,), k, jnp.int32)
                    acc = ws[0] * plsc.load_gather(corners[0], [pt, kv])
                    for c in range(1, 8):
                        acc = acc + ws[c] * plsc.load_gather(corners[c], [pt, kv])
                    plsc.store_scatter(obuf, [pt, kv], acc)
                return 0

            lax.fori_loop(0, NGRP, combine, 0)
            pltpu.sync_copy(obuf, out_hbm.at[pl.ds(pbase, B)])
            return 0

        lax.fori_loop(0, NCHUNK, chunk_body, 0)

    return body(cz, cy, cx, table)


def kernel(coordinates, grid):
    coords = coordinates[0]
    cz = coords[:, 0]
    cy = coords[:, 1]
    cx = coords[:, 2]
    table = jnp.transpose(grid[0].reshape(C, DHW))
    out = _interp_kernel(cz, cy, cx, table)
    return (out,)
